# baseline (speedup 1.0000x reference)
import functools
import math

import jax
import jax.numpy as jnp
from jax.experimental import pallas as pl
from jax.experimental.pallas import tpu as pltpu
from jax.experimental.pallas import tpu_sc as plsc

D_MODEL = 64
SCALE = math.sqrt(D_MODEL)
LANES = 16
WINDOW = 256
PAD_W = 128


def _gather_scale(table128, idx2d, n):
    mesh = plsc.VectorSubcoreMesh(core_axis_name="c", subcore_axis_name="s")

    @functools.partial(
        pl.kernel,
        out_type=jax.ShapeDtypeStruct((n, D_MODEL), jnp.float32),
        mesh=mesh,
        scratch_types=[pltpu.VMEM((WINDOW, PAD_W), jnp.float32)],
    )
    def kern(table_hbm, i_hbm, o_hbm, g_ref):
        def body(i_vmem, o_vmem):
            pltpu.sync_copy(table_hbm.at[i_vmem.at[0]], g_ref)

            @pl.loop(0, WINDOW)
            def _(r):
                for c in range(0, D_MODEL, LANES):
                    src = (pl.ds(r, 1), pl.ds(c, LANES))
                    o_vmem.at[*src][...] = g_ref.at[*src][...] * SCALE

        pltpu.emit_pipeline(
            body,
            grid=(n // WINDOW,),
            in_specs=[pl.BlockSpec((1, WINDOW), lambda i: (0, i))],
            out_specs=[pl.BlockSpec((WINDOW, D_MODEL), lambda i: (i, 0))],
            core_axis_name=("c", "s"),
            dimension_semantics=(pltpu.PARALLEL,),
        )(i_hbm, o_hbm)

    return kern(table128, idx2d)


def kernel(x, table):
    b, s = x.shape
    n = b * s
    idx2d = x.astype(jnp.int32).reshape(1, n)
    table128 = jnp.pad(table, ((0, 0), (0, PAD_W - D_MODEL)))
    out = _gather_scale(table128, idx2d, n)
    return out.reshape(b, s, D_MODEL)
